# baseline (speedup 1.0000x reference)
import functools

import jax
import jax.numpy as jnp
from jax import lax
from jax.experimental import pallas as pl
from jax.experimental.pallas import tpu as pltpu
from jax.experimental.pallas import tpu_sc as plsc

B, S, C = 4096, 200, 7

S_TC = 168
S_SC = S - S_TC
N_TILES = 32

LN2 = 0.6931471805599453
_P5 = (0.04342836333156592, -0.18772049275778527, 0.40871894392121627,
       -0.7057026209301516, 1.4412670742163989, 3.193085771768707e-05)


def _tc_kernel(ch_ref, lab_ref, mask_ref, rew_ref,
               num_ref, corr_ref, nval_ref, accf_ref, acci_ref):
    step = pl.program_id(0)
    nsteps = pl.num_programs(0)

    @pl.when(step == 0)
    def _init():
        accf_ref[0] = 0.0
        accf_ref[1] = 0.0
        acci_ref[0] = 0

    m = mask_ref[...]
    valid = m < 0.5
    vf = valid.astype(jnp.float32)

    loss_part = jnp.float32(0.0)
    correct = valid
    for i in range(3):
        lab = lab_ref[i]
        rew = rew_ref[i]
        ch0 = ch_ref[i, 0]
        mx = ch0
        g = ch0
        for k in range(1, C):
            chk = ch_ref[i, k]
            mx = jnp.maximum(mx, chk)
            g = jnp.where(lab == k, chk, g)
        loss_part += jnp.sum(jnp.log(g) * (rew * vf))
        correct = jnp.logical_and(correct, g >= mx)

    accf_ref[0] += loss_part
    accf_ref[1] += jnp.sum(vf)
    acci_ref[0] += jnp.sum(correct.astype(jnp.int32))

    @pl.when(step == nsteps - 1)
    def _fin():
        num_ref[0] = accf_ref[0]
        corr_ref[0] = acci_ref[0]
        nval_ref[0] = accf_ref[1]


def _ln_sc(x):
    bits = lax.bitcast_convert_type(x, jnp.int32)
    e = lax.shift_right_logical(bits, 23) - 127
    mbits = (bits & 0x007FFFFF) | 0x3F800000
    t = lax.bitcast_convert_type(mbits, jnp.float32) - 1.0
    p = jnp.full((16,), _P5[0], jnp.float32)
    for co in _P5[1:]:
        p = p * t + jnp.float32(co)
    return jnp.float32(LN2) * (e.astype(jnp.float32) + p)


def _sc_kernel(ch_hbm, lab_hbm, mask_hbm, rew_hbm, out_hbm,
               ch_v, lab_v, rew_v, mask_v, buf_v, sem):
    cid = lax.axis_index("c")
    sid = lax.axis_index("s")
    wid = sid * 2 + cid
    col0 = wid * 128
    row0 = S_TC

    copies = []
    for i in range(3):
        for k in range(C):
            copies.append(pltpu.async_copy(
                ch_hbm.at[i, k, pl.ds(row0, S_SC), pl.ds(col0, 128)],
                ch_v.at[i * C + k], sem))
        copies.append(pltpu.async_copy(
            lab_hbm.at[i, pl.ds(row0, S_SC), pl.ds(col0, 128)],
            lab_v.at[i], sem))
        copies.append(pltpu.async_copy(
            rew_hbm.at[i, pl.ds(row0, S_SC), pl.ds(col0, 128)],
            rew_v.at[i], sem))
    copies.append(pltpu.async_copy(
        mask_hbm.at[pl.ds(row0, S_SC), pl.ds(col0, 128)], mask_v, sem))
    for cp in copies:
        cp.wait()

    def body(j, accs):
        acc_l, acc_n, acc_c = accs
        r = j // 8
        cc = (j % 8) * 16
        mv = mask_v[r, pl.ds(cc, 16)]
        valid = mv < 0.5
        vf = jnp.where(valid, 1.0, 0.0).astype(jnp.float32)
        corr = valid
        for i in range(3):
            lab = lab_v[i, r, pl.ds(cc, 16)]
            rew = rew_v[i, r, pl.ds(cc, 16)]
            ch0 = ch_v[i * C, r, pl.ds(cc, 16)]
            mx = ch0
            g = ch0
            for k in range(1, C):
                chk = ch_v[i * C + k, r, pl.ds(cc, 16)]
                mx = jnp.maximum(mx, chk)
                g = jnp.where(lab == k, chk, g)
            acc_l = acc_l + _ln_sc(g) * (rew * vf)
            corr = jnp.logical_and(corr, g >= mx)
        acc_n = acc_n + vf
        acc_c = acc_c + jnp.where(corr, 1.0, 0.0).astype(jnp.float32)
        return (acc_l, acc_n, acc_c)

    z = jnp.zeros((16,), jnp.float32)
    acc_l, acc_n, acc_c = lax.fori_loop(0, S_SC * 8, body, (z, z, z))
    buf_v[0] = acc_l
    buf_v[1] = acc_n
    buf_v[2] = acc_c
    pltpu.sync_copy(buf_v, out_hbm.at[wid])


@functools.partial(jax.jit, static_argnames=("interpret",))
def _impl(output_list, labels_3, mask, reward, interpret):
    chans = jnp.transpose(output_list, (0, 3, 2, 1))
    lab_t = jnp.transpose(labels_3, (2, 1, 0))
    rew_t = jnp.transpose(reward, (2, 1, 0))
    mask_t = mask.T

    sc_call = pl.kernel(
        _sc_kernel,
        mesh=plsc.VectorSubcoreMesh(core_axis_name="c", subcore_axis_name="s"),
        out_type=jax.ShapeDtypeStruct((N_TILES, 3, 16), jnp.float32),
        scratch_types=[
            pltpu.VMEM((3 * C, S_SC, 128), jnp.float32),
            pltpu.VMEM((3, S_SC, 128), jnp.int32),
            pltpu.VMEM((3, S_SC, 128), jnp.float32),
            pltpu.VMEM((S_SC, 128), jnp.float32),
            pltpu.VMEM((3, 16), jnp.float32),
            pltpu.SemaphoreType.DMA,
        ],
    )
    sc_out = sc_call(chans, lab_t, mask_t, rew_t)

    ss = 24
    grid = (S_TC // ss,)
    num, corr, nval = pl.pallas_call(
        _tc_kernel,
        grid=grid,
        in_specs=[
            pl.BlockSpec((3, C, ss, B), lambda j: (0, 0, j, 0)),
            pl.BlockSpec((3, ss, B), lambda j: (0, j, 0)),
            pl.BlockSpec((ss, B), lambda j: (j, 0)),
            pl.BlockSpec((3, ss, B), lambda j: (0, j, 0)),
        ],
        out_specs=[
            pl.BlockSpec(memory_space=pltpu.MemorySpace.SMEM),
            pl.BlockSpec(memory_space=pltpu.MemorySpace.SMEM),
            pl.BlockSpec(memory_space=pltpu.MemorySpace.SMEM),
        ],
        out_shape=[
            jax.ShapeDtypeStruct((1,), jnp.float32),
            jax.ShapeDtypeStruct((1,), jnp.int32),
            jax.ShapeDtypeStruct((1,), jnp.float32),
        ],
        scratch_shapes=[
            pltpu.SMEM((2,), jnp.float32),
            pltpu.SMEM((1,), jnp.int32),
        ],
        interpret=interpret,
    )(chans, lab_t, mask_t, rew_t)

    num_t = num[0] + jnp.sum(sc_out[:, 0, :])
    nval_t = nval[0] + jnp.sum(sc_out[:, 1, :])
    corr_t = corr[0] + jnp.sum(sc_out[:, 2, :]).astype(jnp.int32)
    loss = -num_t / nval_t
    return (loss, corr_t, nval_t.astype(jnp.int32))


def kernel(output_list, labels_3, mask, reward):
    return _impl(output_list, labels_3, mask, reward, False)
